# Initial kernel scaffold; baseline (speedup 1.0000x reference)
import jax
import jax.numpy as jnp
from jax.experimental import pallas as pl


def kernel(entity_ids, relation_ids, negative_sample, entity_embedding, offset_embedding, answer_embedding, translation_mul, translation_add, scaling_mul, scaling_add):
    raise NotImplementedError("write your pallas kernel here")


# speedup vs baseline: 1.8416x; 1.8416x over previous
import functools

import jax
import jax.numpy as jnp
from jax import lax
from jax.experimental import pallas as pl
from jax.experimental.pallas import tpu as pltpu
from jax.experimental.pallas import tpu_sc as plsc

NENTITY = 100000
NREL = 500
DIM = 128
B = 4096
NNEG = 128
GAMMA = 24.0
ALPHA = 0.02

NC, NS = 2, 16
NW = NC * NS
QPW = B // NW
NV = DIM // 16


def _body(eid_hbm, rid_hbm, nidx_hbm, ent_hbm, off_hbm, ans_hbm,
          tmul_hbm, tadd_hbm, smul_hbm, sadd_hbm, out_hbm,
          eid_v, rid_v, nidx_v, cen_v, ofs_v, bufa, bufb, out_v,
          sema, semb, semp):
    wid = lax.axis_index("s") * NC + lax.axis_index("c")
    base = wid * QPW

    with jax.named_scope("stage_idx"):
        pltpu.sync_copy(eid_hbm.at[pl.ds(base, QPW)], eid_v)
        pltpu.sync_copy(rid_hbm.at[pl.ds(base, QPW)], rid_v)
        pltpu.sync_copy(nidx_hbm.at[pl.ds(base, QPW)], nidx_v)

    pltpu.async_copy(ent_hbm.at[eid_v], cen_v, semp)
    pltpu.async_copy(tmul_hbm.at[rid_v], bufa, semp)
    pltpu.async_copy(tadd_hbm.at[rid_v], bufb, semp)
    pltpu.make_async_copy(ent_hbm.at[eid_v], cen_v, semp).wait()
    pltpu.make_async_copy(tmul_hbm.at[rid_v], bufa, semp).wait()
    pltpu.make_async_copy(tadd_hbm.at[rid_v], bufb, semp).wait()

    def cen_body(q, _):
        for j in range(NV):
            sl = pl.ds(16 * j, 16)
            cen_v[q, sl] = cen_v[q, sl] * bufa[q, sl] + bufb[q, sl]
        return _
    lax.fori_loop(0, QPW, cen_body, None)

    pltpu.async_copy(off_hbm.at[eid_v], ofs_v, semp)
    pltpu.async_copy(smul_hbm.at[rid_v], bufa, semp)
    pltpu.async_copy(sadd_hbm.at[rid_v], bufb, semp)
    pltpu.make_async_copy(off_hbm.at[eid_v], ofs_v, semp).wait()
    pltpu.make_async_copy(smul_hbm.at[rid_v], bufa, semp).wait()
    pltpu.make_async_copy(sadd_hbm.at[rid_v], bufb, semp).wait()

    def ofs_body(q, _):
        for j in range(NV):
            sl = pl.ds(16 * j, 16)
            ofs_v[q, sl] = ofs_v[q, sl] * jnp.abs(bufa[q, sl]) + jnp.abs(bufb[q, sl])
        return _
    lax.fori_loop(0, QPW, ofs_body, None)

    C1 = 1.0 - ALPHA

    lane = jnp.arange(16, dtype=jnp.int32)
    rows = [lane + 16 * g for g in range(NNEG // 16)]

    def compute(q, buf):
        qv = jnp.broadcast_to(q, (16,)).astype(jnp.int32)

        zeros = jnp.zeros((16,), jnp.float32)
        init = tuple(zeros for _ in range(2 * (NNEG // 16)))

        @plsc.parallel_loop(0, DIM, unroll=4, carry=init)
        def dbody(d, accs):
            cols = jnp.broadcast_to(d, (16,)).astype(jnp.int32)
            cd = plsc.load_gather(cen_v, [qv, cols])
            od = plsc.load_gather(ofs_v, [qv, cols])
            new = []
            for g in range(NNEG // 16):
                v = plsc.load_gather(buf, [rows[g], cols])
                dd = jnp.abs(v - cd)
                r = jnp.maximum(dd - od, 0.0)
                new.append(accs[2 * g] + r)
                new.append(accs[2 * g + 1] + dd)
            return tuple(new)
        accs = dbody
        for g in range(NNEG // 16):
            out_v[q, pl.ds(16 * g, 16)] = (
                GAMMA - C1 * accs[2 * g] - ALPHA * accs[2 * g + 1])

    pltpu.async_copy(ans_hbm.at[nidx_v.at[0]], bufa, sema)

    def qbody(q2, _):
        q = 2 * q2
        pltpu.async_copy(ans_hbm.at[nidx_v.at[q + 1]], bufb, semb)
        with jax.named_scope("gwait"):
            pltpu.make_async_copy(ans_hbm.at[nidx_v.at[q]], bufa, sema).wait()
        with jax.named_scope("comp"):
            compute(q, bufa)

        @pl.when(q2 < QPW // 2 - 1)
        def _():
            pltpu.async_copy(ans_hbm.at[nidx_v.at[q + 2]], bufa, sema)
        with jax.named_scope("gwait"):
            pltpu.make_async_copy(ans_hbm.at[nidx_v.at[q + 1]], bufb, semb).wait()
        with jax.named_scope("comp"):
            compute(q + 1, bufb)
        return _
    lax.fori_loop(0, QPW // 2, qbody, None)

    pltpu.sync_copy(out_v, out_hbm.at[pl.ds(base, QPW)])


@jax.jit
def _sc_call(eids, rids, nidx, ent, off, ans, tmul, tadd, smul, sadd):
    mesh = plsc.VectorSubcoreMesh(core_axis_name="c", subcore_axis_name="s",
                                  num_cores=NC, num_subcores=NS)
    return pl.kernel(
        _body,
        out_type=jax.ShapeDtypeStruct((B, NNEG), jnp.float32),
        mesh=mesh,
        compiler_params=pltpu.CompilerParams(needs_layout_passes=False),
        scratch_types=[
            pltpu.VMEM((QPW,), jnp.int32),
            pltpu.VMEM((QPW,), jnp.int32),
            pltpu.VMEM((QPW, NNEG), jnp.int32),
            pltpu.VMEM((QPW, DIM), jnp.float32),
            pltpu.VMEM((QPW, DIM), jnp.float32),
            pltpu.VMEM((NNEG, DIM), jnp.float32),
            pltpu.VMEM((NNEG, DIM), jnp.float32),
            pltpu.VMEM((QPW, NNEG), jnp.float32),
            pltpu.SemaphoreType.DMA,
            pltpu.SemaphoreType.DMA,
            pltpu.SemaphoreType.DMA,
        ],
    )(eids, rids, nidx, ent, off, ans, tmul, tadd, smul, sadd)


def kernel(entity_ids, relation_ids, negative_sample, entity_embedding,
           offset_embedding, answer_embedding, translation_mul,
           translation_add, scaling_mul, scaling_add):
    return _sc_call(entity_ids.astype(jnp.int32),
                    relation_ids.astype(jnp.int32),
                    negative_sample.astype(jnp.int32),
                    entity_embedding, offset_embedding, answer_embedding,
                    translation_mul, translation_add, scaling_mul, scaling_add)
